# baseline (speedup 1.0000x reference)
import jax
import jax.numpy as jnp
from jax import lax
from jax.experimental import pallas as pl
from jax.experimental.pallas import tpu as pltpu
from jax.experimental.pallas import tpu_sc as plsc

B = 16384
D = 32
N = 1_000_000
CH = 16384
NFULL = N // CH
TAIL = 512
NTAIL = 64
NCH = NFULL + 1
CPS = NCH // 2
SB = 1024
NVR = SB // 16


def _sweep(idx_hbm, tabT_hbm, tail_hbm, outT_hbm, cid, sid, base,
           idx_v, spm, order_sm, cnt_sm, off_sm, flat_v, dst_v, stage_v,
           tail_v, sem, gsem, gsem2, sem3):
    lanes = jnp.arange(16, dtype=jnp.int32)
    pltpu.sync_copy(tail_hbm, tail_v)

    def zbody(j, _):
        for c in range(D):
            stage_v[c, pl.ds(j * 16, 16)] = jnp.zeros((16,), jnp.float32)
        return 0
    lax.fori_loop(0, SB // 16, zbody, 0)

    pltpu.sync_copy(idx_hbm.at[pl.ds(base, SB)], idx_v)

    for b in range(NCH):
        cnt_sm[b] = 0

    def hbody(j, _):
        v = idx_v[pl.ds(j * 16, 16)]
        for l in range(16):
            b = lax.shift_right_logical(v[l], 14)
            cnt_sm[b] = cnt_sm[b] + 1
        return 0
    lax.fori_loop(0, NVR, hbody, 0)

    run = jnp.int32(0)
    for b in range(NCH):
        off_sm[b] = run
        run = run + cnt_sm[b]
    off_sm[NCH] = run
    for b in range(NCH):
        cnt_sm[b] = off_sm[b]

    def sbody(j, _):
        v = idx_v[pl.ds(j * 16, 16)]
        for l in range(16):
            val = v[l]
            b = lax.shift_right_logical(val, 14)
            o = cnt_sm[b]
            order_sm[o] = (val << 10) | (j * 16 + l)
            cnt_sm[b] = o + 1
        return 0
    lax.fori_loop(0, NVR, sbody, 0)

    row = 2 * sid

    def _fire(g, boff, s):
        @pl.when(g < NFULL)
        def _full():
            pltpu.async_copy(tabT_hbm.at[row, pl.ds(g * CH, CH)],
                             spm.at[pl.ds(boff + row * CH, CH)], s)
            pltpu.async_copy(tabT_hbm.at[row + 1, pl.ds(g * CH, CH)],
                             spm.at[pl.ds(boff + (row + 1) * CH, CH)], s)

        @pl.when(g == NFULL)
        def _tl():
            pltpu.async_copy(tabT_hbm.at[row, pl.ds(NFULL * CH, TAIL)],
                             spm.at[pl.ds(boff + row * CH, TAIL)], s)
            pltpu.async_copy(tabT_hbm.at[row + 1, pl.ds(NFULL * CH, TAIL)],
                             spm.at[pl.ds(boff + (row + 1) * CH, TAIL)], s)

    def _drain(g, boff, s):
        @pl.when(g < NFULL)
        def _full():
            pltpu.make_async_copy(tabT_hbm.at[row, pl.ds(g * CH, CH)],
                                  spm.at[pl.ds(boff + row * CH, CH)],
                                  s).wait()
            pltpu.make_async_copy(tabT_hbm.at[row + 1, pl.ds(g * CH, CH)],
                                  spm.at[pl.ds(boff + (row + 1) * CH, CH)],
                                  s).wait()

        @pl.when(g == NFULL)
        def _tl():
            pltpu.make_async_copy(tabT_hbm.at[row, pl.ds(NFULL * CH, TAIL)],
                                  spm.at[pl.ds(boff + row * CH, TAIL)],
                                  s).wait()
            pltpu.make_async_copy(
                tabT_hbm.at[row + 1, pl.ds(NFULL * CH, TAIL)],
                spm.at[pl.ds(boff + (row + 1) * CH, TAIL)], s).wait()

    _fire(cid * CPS, 0, sem)

    def cbody(gl, _):
        g = cid * CPS + gl
        lo = g * CH
        par = lax.bitwise_and(gl, 1)
        poff = par * (D * CH)
        noff = (1 - par) * (D * CH)

        @pl.when(par == 0)
        def _d0():
            _drain(g, poff, sem)

        @pl.when(par == 1)
        def _d1():
            _drain(g, poff, gsem2)

        plsc.subcore_barrier()

        @pl.when(gl + 1 < CPS)
        def _prefetch():
            @pl.when(par == 0)
            def _p0():
                _fire(g + 1, noff, gsem2)

            @pl.when(par == 1)
            def _p1():
                _fire(g + 1, noff, sem)

        st = off_sm[g]
        n = off_sm[g + 1] - st
        limit = jnp.where(g == NFULL, TAIL, CH)

        def gbody(k, _):
            packed = jnp.zeros((16,), jnp.int32)
            for l in range(16):
                rd = jnp.minimum(st + k * 16 + l, SB - 1)
                pv = order_sm[rd]
                packed = jnp.where(lanes == l,
                                   jnp.full((16,), pv, jnp.int32), packed)
            valid = (k * 16 + lanes) < n
            bse = lax.shift_right_logical(packed, 10) - lo
            pos = lax.bitwise_and(packed, 1023)
            in_spm = valid & (bse < limit)
            for c in range(D):
                flat_v[pl.ds(c * 16, 16)] = jnp.where(
                    in_spm, poff + bse + c * CH, -1)
            cp = pltpu.async_copy(
                spm.at[plsc.Indices(flat_v, ignored_value=-1)], dst_v, gsem)
            cp.wait()
            for c in range(D):
                vals = dst_v[pl.ds(c * 16, 16)]
                plsc.store_scatter(
                    stage_v, [jnp.full((16,), c, jnp.int32), pos],
                    vals, mask=in_spm)

            @pl.when(g == NFULL)
            def _tail_gather():
                mask_t = valid & (bse >= TAIL)
                r_off = lax.bitwise_and(bse - TAIL, NTAIL - 1)
                for c in range(D):
                    vals = plsc.load_gather(
                        tail_v, [r_off, jnp.full((16,), c, jnp.int32)])
                    plsc.store_scatter(
                        stage_v, [jnp.full((16,), c, jnp.int32), pos],
                        vals, mask=mask_t)
            return 0
        lax.fori_loop(0, (n + 15) // 16, gbody, 0)
        return 0

    lax.fori_loop(0, CPS, cbody, 0)
    pltpu.sync_copy(stage_v, outT_hbm.at[cid, :, pl.ds(base, SB)])
    plsc.subcore_barrier()


def _sc_body(uidx_hbm, iidx_hbm, utabT_hbm, itabT_hbm, utail_hbm, itail_hbm,
             uoutT_hbm, ioutT_hbm,
             idx_v, spm, order_sm, cnt_sm, off_sm, flat_v, dst_v, stage_v,
             tail_v, sem, gsem, gsem2, sem3):
    cid = lax.axis_index("c")
    sid = lax.axis_index("s")
    base = sid * SB
    _sweep(uidx_hbm, utabT_hbm, utail_hbm, uoutT_hbm, cid, sid, base,
           idx_v, spm, order_sm, cnt_sm, off_sm, flat_v, dst_v, stage_v,
           tail_v, sem, gsem, gsem2, sem3)
    _sweep(iidx_hbm, itabT_hbm, itail_hbm, ioutT_hbm, cid, sid, base,
           idx_v, spm, order_sm, cnt_sm, off_sm, flat_v, dst_v, stage_v,
           tail_v, sem, gsem, gsem2, sem3)


_sc_gather = pl.kernel(
    _sc_body,
    out_type=(
        jax.ShapeDtypeStruct((2, D, B), jnp.float32),
        jax.ShapeDtypeStruct((2, D, B), jnp.float32),
    ),
    mesh=plsc.VectorSubcoreMesh(core_axis_name="c", subcore_axis_name="s"),
    compiler_params=pltpu.CompilerParams(needs_layout_passes=False),
    scratch_types=(
        pltpu.VMEM((SB,), jnp.int32),
        pltpu.VMEM_SHARED((2 * D * CH,), jnp.float32),
        pltpu.SMEM((SB,), jnp.int32),
        pltpu.SMEM((NCH + 2,), jnp.int32),
        pltpu.SMEM((NCH + 2,), jnp.int32),
        pltpu.VMEM((16 * D,), jnp.int32),
        pltpu.VMEM((16 * D,), jnp.float32),
        pltpu.VMEM((D, SB), jnp.float32),
        pltpu.VMEM((NTAIL, D), jnp.float32),
        pltpu.SemaphoreType.DMA,
        pltpu.SemaphoreType.DMA,
        pltpu.SemaphoreType.DMA,
        pltpu.SemaphoreType.DMA,
    ),
)


BLK = 2048


def _mlp_body(ueT_ref, ieT_ref, w1_ref, b1_ref, w2_ref, b2_ref, w3_ref,
              b3_ref, out_ref):
    u2 = ueT_ref[...]
    i2 = ieT_ref[...]
    xu = u2[0] + u2[1]
    xi = i2[0] + i2[1]
    w1 = w1_ref[...]
    cdims = (((0,), (0,)), ((), ()))
    x1 = (lax.dot_general(w1[:D, :], xu, cdims,
                          preferred_element_type=jnp.float32)
          + lax.dot_general(w1[D:, :], xi, cdims,
                            preferred_element_type=jnp.float32)
          + b1_ref[...])
    h1 = jnp.maximum(x1, 0.0)
    h2 = jnp.maximum(
        lax.dot_general(w2_ref[...], h1, cdims,
                        preferred_element_type=jnp.float32)
        + b2_ref[...], 0.0)
    out_ref[...] = (lax.dot_general(w3_ref[...], h2, cdims,
                                    preferred_element_type=jnp.float32)
                    + b3_ref[...])


def _mlp(ueT2, ieT2, W1, b1, W2, b2, W3, b3):
    grid = (B // BLK,)
    return pl.pallas_call(
        _mlp_body,
        grid=grid,
        in_specs=[
            pl.BlockSpec((2, D, BLK), lambda i: (0, 0, i)),
            pl.BlockSpec((2, D, BLK), lambda i: (0, 0, i)),
            pl.BlockSpec((2 * D, 64), lambda i: (0, 0)),
            pl.BlockSpec((64, 1), lambda i: (0, 0)),
            pl.BlockSpec((64, 16), lambda i: (0, 0)),
            pl.BlockSpec((16, 1), lambda i: (0, 0)),
            pl.BlockSpec((16, 1), lambda i: (0, 0)),
            pl.BlockSpec((1, 1), lambda i: (0, 0)),
        ],
        out_specs=pl.BlockSpec((1, BLK), lambda i: (0, i)),
        out_shape=jax.ShapeDtypeStruct((1, B), jnp.float32),
    )(ueT2, ieT2, W1, b1.reshape(64, 1), W2, b2.reshape(16, 1),
      W3, b3.reshape(1, 1))


def kernel(user_indices, item_indices, user_table, item_table,
           W1, b1, W2, b2, W3, b3):
    utail = lax.slice(user_table, (N - NTAIL, 0), (N, D))
    itail = lax.slice(item_table, (N - NTAIL, 0), (N, D))
    ueT2, ieT2 = _sc_gather(user_indices, item_indices,
                            user_table.T, item_table.T, utail, itail)
    out = _mlp(ueT2, ieT2, W1, b1, W2, b2, W3, b3)
    return out.reshape(B)


# speedup vs baseline: 1.2004x; 1.2004x over previous
import jax
import jax.numpy as jnp
from jax import lax
from jax.experimental import pallas as pl
from jax.experimental.pallas import tpu as pltpu
from jax.experimental.pallas import tpu_sc as plsc

B = 16384
D = 32
N = 1_000_000
CH = 8192
CHSH = 13
NFULL = N // CH
TAIL = 512
NTAIL = 64
NCH = NFULL + 1
CPS0 = 62
SB = 1024
NVR = SB // 16
NBUF = 4


def _sweep(idx_hbm, tabT_hbm, tail_hbm, outT_hbm, cid, sid, base,
           idx_v, spm, order_sm, cnt_sm, off_sm, flat_v, dst_v, stage_v,
           tail_v, sem, gsem, gsem2, sem3, sem4):
    lanes = jnp.arange(16, dtype=jnp.int32)
    pltpu.sync_copy(tail_hbm, tail_v)

    def zbody(j, _):
        for c in range(D):
            stage_v[c, pl.ds(j * 16, 16)] = jnp.zeros((16,), jnp.float32)
        return 0
    lax.fori_loop(0, SB // 16, zbody, 0)

    pltpu.sync_copy(idx_hbm.at[pl.ds(base, SB)], idx_v)

    for b in range(NCH):
        cnt_sm[b] = 0

    def hbody(j, _):
        v = idx_v[pl.ds(j * 16, 16)]
        for l in range(16):
            b = lax.shift_right_logical(v[l], CHSH)
            cnt_sm[b] = cnt_sm[b] + 1
        return 0
    lax.fori_loop(0, NVR, hbody, 0)

    run = jnp.int32(0)
    for b in range(NCH):
        off_sm[b] = run
        run = run + cnt_sm[b]
    off_sm[NCH] = run
    for b in range(NCH):
        cnt_sm[b] = off_sm[b]

    def sbody(j, _):
        v = idx_v[pl.ds(j * 16, 16)]
        for l in range(16):
            val = v[l]
            b = lax.shift_right_logical(val, CHSH)
            o = cnt_sm[b]
            order_sm[o] = (val << 10) | (j * 16 + l)
            cnt_sm[b] = o + 1
        return 0
    lax.fori_loop(0, NVR, sbody, 0)

    row = 2 * sid

    def _fire(g, boff, s):
        @pl.when(g < NFULL)
        def _full():
            pltpu.async_copy(tabT_hbm.at[row, pl.ds(g * CH, CH)],
                             spm.at[pl.ds(boff + row * CH, CH)], s)
            pltpu.async_copy(tabT_hbm.at[row + 1, pl.ds(g * CH, CH)],
                             spm.at[pl.ds(boff + (row + 1) * CH, CH)], s)

        @pl.when(g == NFULL)
        def _tl():
            pltpu.async_copy(tabT_hbm.at[row, pl.ds(NFULL * CH, TAIL)],
                             spm.at[pl.ds(boff + row * CH, TAIL)], s)
            pltpu.async_copy(tabT_hbm.at[row + 1, pl.ds(NFULL * CH, TAIL)],
                             spm.at[pl.ds(boff + (row + 1) * CH, TAIL)], s)

    def _drain(g, boff, s):
        @pl.when(g < NFULL)
        def _full():
            pltpu.make_async_copy(tabT_hbm.at[row, pl.ds(g * CH, CH)],
                                  spm.at[pl.ds(boff + row * CH, CH)],
                                  s).wait()
            pltpu.make_async_copy(tabT_hbm.at[row + 1, pl.ds(g * CH, CH)],
                                  spm.at[pl.ds(boff + (row + 1) * CH, CH)],
                                  s).wait()

        @pl.when(g == NFULL)
        def _tl():
            pltpu.make_async_copy(tabT_hbm.at[row, pl.ds(NFULL * CH, TAIL)],
                                  spm.at[pl.ds(boff + row * CH, TAIL)],
                                  s).wait()
            pltpu.make_async_copy(
                tabT_hbm.at[row + 1, pl.ds(NFULL * CH, TAIL)],
                spm.at[pl.ds(boff + (row + 1) * CH, TAIL)], s).wait()

    g0 = cid * CPS0
    cps = CPS0 - cid
    sems = (sem, gsem2, sem3, sem4)
    _fire(g0, 0, sem)
    _fire(g0 + 1, D * CH, gsem2)

    def cbody(gl, _):
        g = g0 + gl
        lo = g * CH
        par = lax.rem(gl, NBUF)
        npar = lax.rem(gl + 2, NBUF)
        poff = par * (D * CH)
        noff = npar * (D * CH)

        @pl.when(gl + 2 < cps)
        def _prefetch():
            for q in range(NBUF):
                @pl.when(npar == q)
                def _p(q=q):
                    _fire(g + 2, noff, sems[q])

        for q in range(NBUF):
            @pl.when(par == q)
            def _d(q=q):
                _drain(g, poff, sems[q])

        plsc.subcore_barrier()

        st = off_sm[g]
        n = off_sm[g + 1] - st
        limit = jnp.where(g == NFULL, TAIL, CH)

        def gbody(k, _):
            packed = jnp.zeros((16,), jnp.int32)
            for l in range(16):
                rd = jnp.minimum(st + k * 16 + l, SB - 1)
                pv = order_sm[rd]
                packed = jnp.where(lanes == l,
                                   jnp.full((16,), pv, jnp.int32), packed)
            valid = (k * 16 + lanes) < n
            bse = lax.shift_right_logical(packed, 10) - lo
            pos = lax.bitwise_and(packed, 1023)
            in_spm = valid & (bse < limit)
            for c in range(D):
                flat_v[pl.ds(c * 16, 16)] = jnp.where(
                    in_spm, poff + bse + c * CH, -1)
            cp = pltpu.async_copy(
                spm.at[plsc.Indices(flat_v, ignored_value=-1)], dst_v, gsem)
            cp.wait()
            for c in range(D):
                vals = dst_v[pl.ds(c * 16, 16)]
                plsc.store_scatter(
                    stage_v, [jnp.full((16,), c, jnp.int32), pos],
                    vals, mask=in_spm)

            @pl.when(g == NFULL)
            def _tail_gather():
                mask_t = valid & (bse >= TAIL)
                r_off = lax.bitwise_and(bse - TAIL, NTAIL - 1)
                for c in range(D):
                    vals = plsc.load_gather(
                        tail_v, [r_off, jnp.full((16,), c, jnp.int32)])
                    plsc.store_scatter(
                        stage_v, [jnp.full((16,), c, jnp.int32), pos],
                        vals, mask=mask_t)
            return 0
        lax.fori_loop(0, (n + 15) // 16, gbody, 0)
        return 0

    lax.fori_loop(0, cps, cbody, 0)
    pltpu.sync_copy(stage_v, outT_hbm.at[cid, :, pl.ds(base, SB)])
    plsc.subcore_barrier()


def _sc_body(uidx_hbm, iidx_hbm, utabT_hbm, itabT_hbm, utail_hbm, itail_hbm,
             uoutT_hbm, ioutT_hbm,
             idx_v, spm, order_sm, cnt_sm, off_sm, flat_v, dst_v, stage_v,
             tail_v, sem, gsem, gsem2, sem3, sem4):
    cid = lax.axis_index("c")
    sid = lax.axis_index("s")
    base = sid * SB
    _sweep(uidx_hbm, utabT_hbm, utail_hbm, uoutT_hbm, cid, sid, base,
           idx_v, spm, order_sm, cnt_sm, off_sm, flat_v, dst_v, stage_v,
           tail_v, sem, gsem, gsem2, sem3, sem4)
    _sweep(iidx_hbm, itabT_hbm, itail_hbm, ioutT_hbm, cid, sid, base,
           idx_v, spm, order_sm, cnt_sm, off_sm, flat_v, dst_v, stage_v,
           tail_v, sem, gsem, gsem2, sem3, sem4)


_sc_gather = pl.kernel(
    _sc_body,
    out_type=(
        jax.ShapeDtypeStruct((2, D, B), jnp.float32),
        jax.ShapeDtypeStruct((2, D, B), jnp.float32),
    ),
    mesh=plsc.VectorSubcoreMesh(core_axis_name="c", subcore_axis_name="s"),
    compiler_params=pltpu.CompilerParams(needs_layout_passes=False),
    scratch_types=(
        pltpu.VMEM((SB,), jnp.int32),
        pltpu.VMEM_SHARED((NBUF * D * CH,), jnp.float32),
        pltpu.SMEM((SB,), jnp.int32),
        pltpu.SMEM((NCH + 2,), jnp.int32),
        pltpu.SMEM((NCH + 2,), jnp.int32),
        pltpu.VMEM((16 * D,), jnp.int32),
        pltpu.VMEM((16 * D,), jnp.float32),
        pltpu.VMEM((D, SB), jnp.float32),
        pltpu.VMEM((NTAIL, D), jnp.float32),
        pltpu.SemaphoreType.DMA,
        pltpu.SemaphoreType.DMA,
        pltpu.SemaphoreType.DMA,
        pltpu.SemaphoreType.DMA,
        pltpu.SemaphoreType.DMA,
    ),
)


BLK = 2048


def _mlp_body(ueT_ref, ieT_ref, w1_ref, b1_ref, w2_ref, b2_ref, w3_ref,
              b3_ref, out_ref):
    u2 = ueT_ref[...]
    i2 = ieT_ref[...]
    xu = u2[0] + u2[1]
    xi = i2[0] + i2[1]
    w1 = w1_ref[...]
    cdims = (((0,), (0,)), ((), ()))
    x1 = (lax.dot_general(w1[:D, :], xu, cdims,
                          preferred_element_type=jnp.float32)
          + lax.dot_general(w1[D:, :], xi, cdims,
                            preferred_element_type=jnp.float32)
          + b1_ref[...])
    h1 = jnp.maximum(x1, 0.0)
    h2 = jnp.maximum(
        lax.dot_general(w2_ref[...], h1, cdims,
                        preferred_element_type=jnp.float32)
        + b2_ref[...], 0.0)
    out_ref[...] = (lax.dot_general(w3_ref[...], h2, cdims,
                                    preferred_element_type=jnp.float32)
                    + b3_ref[...])


def _mlp(ueT2, ieT2, W1, b1, W2, b2, W3, b3):
    grid = (B // BLK,)
    return pl.pallas_call(
        _mlp_body,
        grid=grid,
        in_specs=[
            pl.BlockSpec((2, D, BLK), lambda i: (0, 0, i)),
            pl.BlockSpec((2, D, BLK), lambda i: (0, 0, i)),
            pl.BlockSpec((2 * D, 64), lambda i: (0, 0)),
            pl.BlockSpec((64, 1), lambda i: (0, 0)),
            pl.BlockSpec((64, 16), lambda i: (0, 0)),
            pl.BlockSpec((16, 1), lambda i: (0, 0)),
            pl.BlockSpec((16, 1), lambda i: (0, 0)),
            pl.BlockSpec((1, 1), lambda i: (0, 0)),
        ],
        out_specs=pl.BlockSpec((1, BLK), lambda i: (0, i)),
        out_shape=jax.ShapeDtypeStruct((1, B), jnp.float32),
    )(ueT2, ieT2, W1, b1.reshape(64, 1), W2, b2.reshape(16, 1),
      W3, b3.reshape(1, 1))


def kernel(user_indices, item_indices, user_table, item_table,
           W1, b1, W2, b2, W3, b3):
    utail = lax.slice(user_table, (N - NTAIL, 0), (N, D))
    itail = lax.slice(item_table, (N - NTAIL, 0), (N, D))
    ueT2, ieT2 = _sc_gather(user_indices, item_indices,
                            user_table.T, item_table.T, utail, itail)
    out = _mlp(ueT2, ieT2, W1, b1, W2, b2, W3, b3)
    return out.reshape(B)


# speedup vs baseline: 1.2767x; 1.0636x over previous
import jax
import jax.numpy as jnp
from jax import lax
from jax.experimental import pallas as pl
from jax.experimental.pallas import tpu as pltpu
from jax.experimental.pallas import tpu_sc as plsc

B = 16384
D = 32
N = 1_000_000
CH = 8192
CHSH = 13
NFULL = N // CH
TAIL = 512
NTAIL = 64
NCH = NFULL + 1
CPS0 = 62
SB = 1024
NVR = SB // 16
NBUF = 4


def _sweep(idx_hbm, tabT_hbm, tail_hbm, outT_hbm, cid, sid, base,
           idx_v, spm, order_v, counts2d, off_sm, flat_v, dst_v, stage_v,
           tail_v, sem, gsem, gsem2, sem3, sem4):
    lanes = jnp.arange(16, dtype=jnp.int32)
    ones = jnp.ones((16,), jnp.int32)
    pltpu.sync_copy(tail_hbm, tail_v)

    def zbody(j, _):
        for c in range(D):
            stage_v[c, pl.ds(j * 16, 16)] = jnp.zeros((16,), jnp.float32)
        return 0
    lax.fori_loop(0, SB // 16, zbody, 0)

    pltpu.sync_copy(idx_hbm.at[pl.ds(base, SB)], idx_v)

    def z2body(b, _):
        counts2d[b] = jnp.zeros((16,), jnp.int32)
        return 0
    lax.fori_loop(0, NCH, z2body, 0)

    def hbody(j, _):
        v = idx_v[pl.ds(j * 16, 16)]
        bv = lax.shift_right_logical(v, CHSH)
        plsc.addupdate_scatter(counts2d, [bv, lanes], ones)
        return 0
    lax.fori_loop(0, NVR, hbody, 0)

    def pbody(b, carry):
        v = counts2d[b]
        cum = plsc.cumsum(v)
        excl = cum - v + carry
        counts2d[b] = excl
        off_sm[b] = excl[0]
        return carry + cum[15]
    lax.fori_loop(0, NCH, pbody, jnp.int32(0))
    off_sm[NCH] = SB

    def sbody(j, _):
        v = idx_v[pl.ds(j * 16, 16)]
        bv = lax.shift_right_logical(v, CHSH)
        packed = (v << 10) | (j * 16 + lanes)
        ov = plsc.load_gather(counts2d, [bv, lanes])
        plsc.store_scatter(order_v, [ov], packed)
        plsc.addupdate_scatter(counts2d, [bv, lanes], ones)
        return 0
    lax.fori_loop(0, NVR, sbody, 0)

    row = 2 * sid

    def _fire(g, boff, s):
        @pl.when(g < NFULL)
        def _full():
            pltpu.async_copy(tabT_hbm.at[row, pl.ds(g * CH, CH)],
                             spm.at[pl.ds(boff + row * CH, CH)], s)
            pltpu.async_copy(tabT_hbm.at[row + 1, pl.ds(g * CH, CH)],
                             spm.at[pl.ds(boff + (row + 1) * CH, CH)], s)

        @pl.when(g == NFULL)
        def _tl():
            pltpu.async_copy(tabT_hbm.at[row, pl.ds(NFULL * CH, TAIL)],
                             spm.at[pl.ds(boff + row * CH, TAIL)], s)
            pltpu.async_copy(tabT_hbm.at[row + 1, pl.ds(NFULL * CH, TAIL)],
                             spm.at[pl.ds(boff + (row + 1) * CH, TAIL)], s)

    def _drain(g, boff, s):
        @pl.when(g < NFULL)
        def _full():
            pltpu.make_async_copy(tabT_hbm.at[row, pl.ds(g * CH, CH)],
                                  spm.at[pl.ds(boff + row * CH, CH)],
                                  s).wait()
            pltpu.make_async_copy(tabT_hbm.at[row + 1, pl.ds(g * CH, CH)],
                                  spm.at[pl.ds(boff + (row + 1) * CH, CH)],
                                  s).wait()

        @pl.when(g == NFULL)
        def _tl():
            pltpu.make_async_copy(tabT_hbm.at[row, pl.ds(NFULL * CH, TAIL)],
                                  spm.at[pl.ds(boff + row * CH, TAIL)],
                                  s).wait()
            pltpu.make_async_copy(
                tabT_hbm.at[row + 1, pl.ds(NFULL * CH, TAIL)],
                spm.at[pl.ds(boff + (row + 1) * CH, TAIL)], s).wait()

    g0 = cid * CPS0
    cps = CPS0 - cid
    sems = (sem, gsem2, sem3, sem4)
    _fire(g0, 0, sem)
    _fire(g0 + 1, D * CH, gsem2)

    def cbody(gl, _):
        g = g0 + gl
        lo = g * CH
        par = lax.rem(gl, NBUF)
        npar = lax.rem(gl + 2, NBUF)
        poff = par * (D * CH)
        noff = npar * (D * CH)

        @pl.when(gl + 2 < cps)
        def _prefetch():
            for q in range(NBUF):
                @pl.when(npar == q)
                def _p(q=q):
                    _fire(g + 2, noff, sems[q])

        for q in range(NBUF):
            @pl.when(par == q)
            def _d(q=q):
                _drain(g, poff, sems[q])

        plsc.subcore_barrier()

        st = off_sm[g]
        n = off_sm[g + 1] - st
        limit = jnp.where(g == NFULL, TAIL, CH)

        def gbody(k, _):
            packed = order_v[pl.ds(st + k * 16, 16)]
            valid = (k * 16 + lanes) < n
            bse = lax.shift_right_logical(packed, 10) - lo
            pos = lax.bitwise_and(packed, 1023)
            in_spm = valid & (bse < limit)
            for c in range(D):
                flat_v[pl.ds(c * 16, 16)] = jnp.where(
                    in_spm, poff + bse + c * CH, -1)
            cp = pltpu.async_copy(
                spm.at[plsc.Indices(flat_v, ignored_value=-1)], dst_v, gsem)
            cp.wait()
            for c in range(D):
                vals = dst_v[pl.ds(c * 16, 16)]
                plsc.store_scatter(
                    stage_v, [jnp.full((16,), c, jnp.int32), pos],
                    vals, mask=in_spm)

            @pl.when(g == NFULL)
            def _tail_gather():
                mask_t = valid & (bse >= TAIL)
                r_off = lax.bitwise_and(bse - TAIL, NTAIL - 1)
                for c in range(D):
                    vals = plsc.load_gather(
                        tail_v, [r_off, jnp.full((16,), c, jnp.int32)])
                    plsc.store_scatter(
                        stage_v, [jnp.full((16,), c, jnp.int32), pos],
                        vals, mask=mask_t)
            return 0
        lax.fori_loop(0, (n + 15) // 16, gbody, 0)
        return 0

    lax.fori_loop(0, cps, cbody, 0)
    pltpu.sync_copy(stage_v, outT_hbm.at[cid, :, pl.ds(base, SB)])
    plsc.subcore_barrier()


def _sc_body(uidx_hbm, iidx_hbm, utabT_hbm, itabT_hbm, utail_hbm, itail_hbm,
             uoutT_hbm, ioutT_hbm,
             idx_v, spm, order_v, counts2d, off_sm, flat_v, dst_v, stage_v,
             tail_v, sem, gsem, gsem2, sem3, sem4):
    cid = lax.axis_index("c")
    sid = lax.axis_index("s")
    base = sid * SB
    _sweep(uidx_hbm, utabT_hbm, utail_hbm, uoutT_hbm, cid, sid, base,
           idx_v, spm, order_v, counts2d, off_sm, flat_v, dst_v, stage_v,
           tail_v, sem, gsem, gsem2, sem3, sem4)
    _sweep(iidx_hbm, itabT_hbm, itail_hbm, ioutT_hbm, cid, sid, base,
           idx_v, spm, order_v, counts2d, off_sm, flat_v, dst_v, stage_v,
           tail_v, sem, gsem, gsem2, sem3, sem4)


_sc_gather = pl.kernel(
    _sc_body,
    out_type=(
        jax.ShapeDtypeStruct((2, D, B), jnp.float32),
        jax.ShapeDtypeStruct((2, D, B), jnp.float32),
    ),
    mesh=plsc.VectorSubcoreMesh(core_axis_name="c", subcore_axis_name="s"),
    compiler_params=pltpu.CompilerParams(needs_layout_passes=False),
    scratch_types=(
        pltpu.VMEM((SB,), jnp.int32),
        pltpu.VMEM_SHARED((NBUF * D * CH,), jnp.float32),
        pltpu.VMEM((SB + 16,), jnp.int32),
        pltpu.VMEM((NCH, 16), jnp.int32),
        pltpu.SMEM((NCH + 2,), jnp.int32),
        pltpu.VMEM((16 * D,), jnp.int32),
        pltpu.VMEM((16 * D,), jnp.float32),
        pltpu.VMEM((D, SB), jnp.float32),
        pltpu.VMEM((NTAIL, D), jnp.float32),
        pltpu.SemaphoreType.DMA,
        pltpu.SemaphoreType.DMA,
        pltpu.SemaphoreType.DMA,
        pltpu.SemaphoreType.DMA,
        pltpu.SemaphoreType.DMA,
    ),
)


BLK = 2048


def _mlp_body(ueT_ref, ieT_ref, w1_ref, b1_ref, w2_ref, b2_ref, w3_ref,
              b3_ref, out_ref):
    u2 = ueT_ref[...]
    i2 = ieT_ref[...]
    xu = u2[0] + u2[1]
    xi = i2[0] + i2[1]
    w1 = w1_ref[...]
    cdims = (((0,), (0,)), ((), ()))
    x1 = (lax.dot_general(w1[:D, :], xu, cdims,
                          preferred_element_type=jnp.float32)
          + lax.dot_general(w1[D:, :], xi, cdims,
                            preferred_element_type=jnp.float32)
          + b1_ref[...])
    h1 = jnp.maximum(x1, 0.0)
    h2 = jnp.maximum(
        lax.dot_general(w2_ref[...], h1, cdims,
                        preferred_element_type=jnp.float32)
        + b2_ref[...], 0.0)
    out_ref[...] = (lax.dot_general(w3_ref[...], h2, cdims,
                                    preferred_element_type=jnp.float32)
                    + b3_ref[...])


def _mlp(ueT2, ieT2, W1, b1, W2, b2, W3, b3):
    grid = (B // BLK,)
    return pl.pallas_call(
        _mlp_body,
        grid=grid,
        in_specs=[
            pl.BlockSpec((2, D, BLK), lambda i: (0, 0, i)),
            pl.BlockSpec((2, D, BLK), lambda i: (0, 0, i)),
            pl.BlockSpec((2 * D, 64), lambda i: (0, 0)),
            pl.BlockSpec((64, 1), lambda i: (0, 0)),
            pl.BlockSpec((64, 16), lambda i: (0, 0)),
            pl.BlockSpec((16, 1), lambda i: (0, 0)),
            pl.BlockSpec((16, 1), lambda i: (0, 0)),
            pl.BlockSpec((1, 1), lambda i: (0, 0)),
        ],
        out_specs=pl.BlockSpec((1, BLK), lambda i: (0, i)),
        out_shape=jax.ShapeDtypeStruct((1, B), jnp.float32),
    )(ueT2, ieT2, W1, b1.reshape(64, 1), W2, b2.reshape(16, 1),
      W3, b3.reshape(1, 1))


def kernel(user_indices, item_indices, user_table, item_table,
           W1, b1, W2, b2, W3, b3):
    utail = lax.slice(user_table, (N - NTAIL, 0), (N, D))
    itail = lax.slice(item_table, (N - NTAIL, 0), (N, D))
    ueT2, ieT2 = _sc_gather(user_indices, item_indices,
                            user_table.T, item_table.T, utail, itail)
    out = _mlp(ueT2, ieT2, W1, b1, W2, b2, W3, b3)
    return out.reshape(B)
